# Initial kernel scaffold; baseline (speedup 1.0000x reference)
import jax
import jax.numpy as jnp
from jax.experimental import pallas as pl


def kernel(x, compress_neurons, expand_neurons, Wq, Wk, Wv, Wo):
    raise NotImplementedError("write your pallas kernel here")


# speedup vs baseline: 1.6629x; 1.6629x over previous
import functools

import numpy as np
import jax
import jax.numpy as jnp
from jax import lax
from jax.experimental import pallas as pl
from jax.experimental.pallas import tpu as pltpu

S_BLK = 256


def _topk_cw(scores, nk):
    n = scores.shape[-1]
    iota = lax.broadcasted_iota(jnp.int32, scores.shape, 1)
    work = scores
    masks, tops = [], []
    for _ in range(nk):
        mk = jnp.max(work, axis=-1, keepdims=True)
        elig = work >= mk
        idx = jnp.min(jnp.where(elig, iota, n), axis=-1, keepdims=True)
        mask = iota == idx
        masks.append(mask)
        tops.append(mk)
        work = jnp.where(mask, -jnp.inf, work)
    m1 = tops[0]
    es = [jnp.exp(t - m1) for t in tops]
    denom = sum(es)
    cw = jnp.zeros_like(scores)
    for mask, e in zip(masks, es):
        cw = cw + jnp.where(mask, e / denom, 0.0)
    return cw


def _routing_body(x_ref, wt_ref, cw_ref, *, nk, n_routers, n_experts):
    scores = lax.dot_general(x_ref[...], wt_ref[...], (((1,), (1,)), ((), ())),
                             preferred_element_type=jnp.float32)
    for r in range(n_routers):
        s = scores[:, r * n_experts:(r + 1) * n_experts]
        cw_ref[r] = _topk_cw(s, nk)


def _cproj_body(x_ref, nb_ref, cw_ref, q_ref, k_ref, v_ref, *, epb, rank):
    proj = jnp.dot(x_ref[...], nb_ref[...], preferred_element_type=jnp.float32)

    @pl.when(pl.program_id(1) == 0)
    def _():
        q_ref[...] = jnp.zeros_like(q_ref)
        k_ref[...] = jnp.zeros_like(k_ref)
        v_ref[...] = jnp.zeros_like(v_ref)

    accs = [q_ref[...], k_ref[...], v_ref[...]]
    for i in range(epb):
        p = proj[:, i * rank:(i + 1) * rank]
        for r in range(3):
            accs[r] = accs[r] + cw_ref[r, i, :][:, None] * p
    q_ref[...], k_ref[...], v_ref[...] = accs


def _attn_body(q_ref, k_ref, v_ref, o_ref, *, n_heads, d_head, kc):
    qb = q_ref[...].astype(jnp.bfloat16)
    kb = k_ref[...].astype(jnp.bfloat16)
    v = v_ref[...]
    rows = q_ref.shape[0]
    nchunks = k_ref.shape[0] // kc
    scale = np.float32(1.0 / np.sqrt(d_head))
    outs = []
    for h in range(n_heads):
        qh = qb[:, h * d_head:(h + 1) * d_head]
        m = jnp.full((rows, 1), -jnp.inf, jnp.float32)
        l = jnp.zeros((rows, 1), jnp.float32)
        o = jnp.zeros((rows, d_head), jnp.float32)
        for j in range(nchunks):
            khc = kb[j * kc:(j + 1) * kc, h * d_head:(h + 1) * d_head]
            vhc = v[j * kc:(j + 1) * kc, h * d_head:(h + 1) * d_head]
            s = lax.dot_general(qh, khc, (((1,), (1,)), ((), ())),
                                preferred_element_type=jnp.float32) * scale
            mrow = jnp.max(s, axis=-1, keepdims=True)
            mnew = jnp.maximum(m, mrow)
            corr = jnp.where(m == mnew, np.float32(0.0), m - mnew)
            p = jnp.exp(s - mnew)
            ec = jnp.exp(corr)
            lnew = ec * l + jnp.sum(p, axis=-1, keepdims=True)
            oacc = (ec * l) * o
            omm = oacc + jnp.dot(p, vhc, preferred_element_type=jnp.float32)
            o = omm * (np.float32(1.0) / lnew)
            m, l = mnew, lnew
        outs.append(o)
    o_ref[...] = jnp.concatenate(outs, axis=1)


def _eproj_body(a_ref, nb_ref, cw_ref, o_ref, *, epb, d_model):
    proj = jnp.dot(a_ref[...], nb_ref[...], preferred_element_type=jnp.float32)

    @pl.when(pl.program_id(1) == 0)
    def _():
        o_ref[...] = jnp.zeros_like(o_ref)

    acc = o_ref[...]
    for i in range(epb):
        p = proj[:, i * d_model:(i + 1) * d_model]
        acc = acc + cw_ref[0, i, :][:, None] * p
    o_ref[...] = acc


def kernel(x, compress_neurons, expand_neurons, Wq, Wk, Wv, Wo):
    B, S, D = x.shape
    N, _, R = compress_neurons.shape
    NE = expand_neurons.shape[0]
    n_heads = 12
    d_head = R // n_heads
    xs = x.reshape(B * S, D)
    n_sb = (B * S) // S_BLK

    wt = jnp.concatenate([Wq, Wk, Wv], axis=0)
    cw = pl.pallas_call(
        functools.partial(_routing_body, nk=4, n_routers=3, n_experts=N),
        grid=(n_sb,),
        in_specs=[
            pl.BlockSpec((S_BLK, D), lambda s: (s, 0)),
            pl.BlockSpec((3 * N, D), lambda s: (0, 0)),
        ],
        out_specs=pl.BlockSpec((3, S_BLK, N), lambda s: (0, s, 0)),
        out_shape=jax.ShapeDtypeStruct((3, B * S, N), jnp.float32),
    )(xs, wt)
    cw_t = cw.transpose(0, 2, 1)

    EPB = 16
    nT = compress_neurons.transpose(1, 0, 2).reshape(D, N * R)
    qkv_spec = pl.BlockSpec((S_BLK, R), lambda s, n: (s, 0))
    q, k, v = pl.pallas_call(
        functools.partial(_cproj_body, epb=EPB, rank=R),
        grid=(n_sb, N // EPB),
        in_specs=[
            pl.BlockSpec((S_BLK, D), lambda s, n: (s, 0)),
            pl.BlockSpec((D, EPB * R), lambda s, n: (0, n)),
            pl.BlockSpec((3, EPB, S_BLK), lambda s, n: (0, n, s)),
        ],
        out_specs=[qkv_spec, qkv_spec, qkv_spec],
        out_shape=[jax.ShapeDtypeStruct((B * S, R), jnp.float32)] * 3,
        compiler_params=pltpu.CompilerParams(
            dimension_semantics=("parallel", "arbitrary")),
    )(xs, nT, cw_t)

    attn_out = pl.pallas_call(
        functools.partial(_attn_body, n_heads=n_heads, d_head=d_head, kc=1024),
        grid=(n_sb,),
        in_specs=[
            pl.BlockSpec((S_BLK, R), lambda s: (s, 0)),
            pl.BlockSpec((B * S, R), lambda s: (0, 0)),
            pl.BlockSpec((B * S, R), lambda s: (0, 0)),
        ],
        out_specs=pl.BlockSpec((S_BLK, R), lambda s: (s, 0)),
        out_shape=jax.ShapeDtypeStruct((B * S, R), jnp.float32),
    )(q, k, v)

    cwo = pl.pallas_call(
        functools.partial(_routing_body, nk=2, n_routers=1, n_experts=NE),
        grid=(n_sb,),
        in_specs=[
            pl.BlockSpec((S_BLK, R), lambda s: (s, 0)),
            pl.BlockSpec((NE, R), lambda s: (0, 0)),
        ],
        out_specs=pl.BlockSpec((1, S_BLK, NE), lambda s: (0, s, 0)),
        out_shape=jax.ShapeDtypeStruct((1, B * S, NE), jnp.float32),
    )(attn_out, Wo)
    cwo_t = cwo.transpose(0, 2, 1)

    EPBE = 8
    eT = expand_neurons.transpose(1, 0, 2).reshape(R, NE * D)
    out = pl.pallas_call(
        functools.partial(_eproj_body, epb=EPBE, d_model=D),
        grid=(n_sb, NE // EPBE),
        in_specs=[
            pl.BlockSpec((S_BLK, R), lambda s, n: (s, 0)),
            pl.BlockSpec((R, EPBE * D), lambda s, n: (0, n)),
            pl.BlockSpec((1, EPBE, S_BLK), lambda s, n: (0, n, s)),
        ],
        out_specs=pl.BlockSpec((S_BLK, D), lambda s, n: (s, 0)),
        out_shape=jax.ShapeDtypeStruct((B * S, D), jnp.float32),
        compiler_params=pltpu.CompilerParams(
            dimension_semantics=("parallel", "arbitrary")),
    )(attn_out, eT, cwo_t)

    return out.reshape(B, S, D)


# speedup vs baseline: 2.0970x; 1.2611x over previous
import functools

import numpy as np
import jax
import jax.numpy as jnp
from jax import lax
from jax.experimental import pallas as pl
from jax.experimental.pallas import tpu as pltpu

S_BLK = 256


def _topk_cw(scores, nk):
    n = scores.shape[-1]
    iota = lax.broadcasted_iota(jnp.int32, scores.shape, 1)
    work = scores
    masks, tops = [], []
    for _ in range(nk):
        mk = jnp.max(work, axis=-1, keepdims=True)
        elig = work >= mk
        idx = jnp.min(jnp.where(elig, iota, n), axis=-1, keepdims=True)
        mask = iota == idx
        masks.append(mask)
        tops.append(mk)
        work = jnp.where(mask, -jnp.inf, work)
    m1 = tops[0]
    es = [jnp.exp(t - m1) for t in tops]
    denom = sum(es)
    cw = jnp.zeros_like(scores)
    for mask, e in zip(masks, es):
        cw = cw + jnp.where(mask, e / denom, 0.0)
    return cw


def _routing_body(x_ref, wt_ref, cw_ref, *, nk, n_routers, n_experts):
    scores = lax.dot_general(x_ref[...], wt_ref[...], (((1,), (1,)), ((), ())),
                             preferred_element_type=jnp.float32)
    for r in range(n_routers):
        s = scores[:, r * n_experts:(r + 1) * n_experts]
        cw_ref[r] = _topk_cw(s, nk)


def _cproj_body(x_ref, nb_ref, cw_ref, q_ref, k_ref, v_ref, *, n_experts, rp):
    proj = jnp.dot(x_ref[...], nb_ref[...], preferred_element_type=jnp.float32)
    rows = x_ref.shape[0]
    accs = [jnp.zeros((rows, rp), jnp.float32) for _ in range(3)]
    for e in range(n_experts):
        p = proj[:, e * rp:(e + 1) * rp]
        for r in range(3):
            accs[r] = accs[r] + cw_ref[r, :, e:e + 1] * p
    q_ref[...] = accs[0]
    k_ref[...] = accs[1]
    v_ref[...] = accs[2]


def _attn_body(q_ref, k_ref, v_ref, o_ref, *, n_heads, d_head, kc):
    qb = q_ref[...].astype(jnp.bfloat16)
    kb = k_ref[...].astype(jnp.bfloat16)
    v = v_ref[...]
    rows = q_ref.shape[0]
    nchunks = k_ref.shape[0] // kc
    scale = np.float32(1.0 / np.sqrt(d_head))
    outs = []
    for h in range(n_heads):
        qh = qb[:, h * d_head:(h + 1) * d_head]
        m = jnp.full((rows, 1), -jnp.inf, jnp.float32)
        l = jnp.zeros((rows, 1), jnp.float32)
        o = jnp.zeros((rows, d_head), jnp.float32)
        for j in range(nchunks):
            khc = kb[j * kc:(j + 1) * kc, h * d_head:(h + 1) * d_head]
            vhc = v[j * kc:(j + 1) * kc, h * d_head:(h + 1) * d_head]
            s = lax.dot_general(qh, khc, (((1,), (1,)), ((), ())),
                                preferred_element_type=jnp.float32) * scale
            mrow = jnp.max(s, axis=-1, keepdims=True)
            mnew = jnp.maximum(m, mrow)
            corr = jnp.where(m == mnew, np.float32(0.0), m - mnew)
            p = jnp.exp(s - mnew)
            ec = jnp.exp(corr)
            lnew = ec * l + jnp.sum(p, axis=-1, keepdims=True)
            oacc = (ec * l) * o
            omm = oacc + jnp.dot(p, vhc, preferred_element_type=jnp.float32)
            o = omm * (np.float32(1.0) / lnew)
            m, l = mnew, lnew
        outs.append(o)
    o_ref[...] = jnp.concatenate(outs, axis=1)


def _eproj_body(a_ref, nb_ref, cw_ref, o_ref, *, epb, d_model):
    proj = jnp.dot(a_ref[...], nb_ref[...], preferred_element_type=jnp.float32)

    @pl.when(pl.program_id(1) == 0)
    def _():
        o_ref[...] = jnp.zeros_like(o_ref)

    acc = o_ref[...]
    for i in range(epb):
        p = proj[:, i * d_model:(i + 1) * d_model]
        acc = acc + cw_ref[0, i, :][:, None] * p
    o_ref[...] = acc


def kernel(x, compress_neurons, expand_neurons, Wq, Wk, Wv, Wo):
    B, S, D = x.shape
    N, _, R = compress_neurons.shape
    NE = expand_neurons.shape[0]
    n_heads = 12
    d_head = R // n_heads
    xs = x.reshape(B * S, D)
    n_sb = (B * S) // S_BLK

    wt = jnp.concatenate([Wq, Wk, Wv], axis=0)
    cw = pl.pallas_call(
        functools.partial(_routing_body, nk=4, n_routers=3, n_experts=N),
        grid=(n_sb,),
        in_specs=[
            pl.BlockSpec((S_BLK, D), lambda s: (s, 0)),
            pl.BlockSpec((3 * N, D), lambda s: (0, 0)),
        ],
        out_specs=pl.BlockSpec((3, S_BLK, N), lambda s: (0, s, 0)),
        out_shape=jax.ShapeDtypeStruct((3, B * S, N), jnp.float32),
    )(xs, wt)

    RP = 128
    nT = jnp.pad(compress_neurons.transpose(1, 0, 2), ((0, 0), (0, 0), (0, RP - R))
                 ).reshape(D, N * RP)
    qkv_spec = pl.BlockSpec((S_BLK, RP), lambda s: (s, 0))
    qp, kp, vp = pl.pallas_call(
        functools.partial(_cproj_body, n_experts=N, rp=RP),
        grid=(n_sb,),
        in_specs=[
            pl.BlockSpec((S_BLK, D), lambda s: (s, 0)),
            pl.BlockSpec((D, N * RP), lambda s: (0, 0)),
            pl.BlockSpec((3, S_BLK, N), lambda s: (0, s, 0)),
        ],
        out_specs=[qkv_spec, qkv_spec, qkv_spec],
        out_shape=[jax.ShapeDtypeStruct((B * S, RP), jnp.float32)] * 3,
    )(xs, nT, cw)
    q, k, v = qp[:, :R], kp[:, :R], vp[:, :R]

    attn_out = pl.pallas_call(
        functools.partial(_attn_body, n_heads=n_heads, d_head=d_head, kc=1024),
        grid=(n_sb,),
        in_specs=[
            pl.BlockSpec((S_BLK, R), lambda s: (s, 0)),
            pl.BlockSpec((B * S, R), lambda s: (0, 0)),
            pl.BlockSpec((B * S, R), lambda s: (0, 0)),
        ],
        out_specs=pl.BlockSpec((S_BLK, R), lambda s: (s, 0)),
        out_shape=jax.ShapeDtypeStruct((B * S, R), jnp.float32),
    )(q, k, v)

    cwo = pl.pallas_call(
        functools.partial(_routing_body, nk=2, n_routers=1, n_experts=NE),
        grid=(n_sb,),
        in_specs=[
            pl.BlockSpec((S_BLK, R), lambda s: (s, 0)),
            pl.BlockSpec((NE, R), lambda s: (0, 0)),
        ],
        out_specs=pl.BlockSpec((1, S_BLK, NE), lambda s: (0, s, 0)),
        out_shape=jax.ShapeDtypeStruct((1, B * S, NE), jnp.float32),
    )(attn_out, Wo)
    cwo_t = cwo.transpose(0, 2, 1)

    EPBE = 8
    eT = expand_neurons.transpose(1, 0, 2).reshape(R, NE * D)
    out = pl.pallas_call(
        functools.partial(_eproj_body, epb=EPBE, d_model=D),
        grid=(n_sb, NE // EPBE),
        in_specs=[
            pl.BlockSpec((S_BLK, R), lambda s, n: (s, 0)),
            pl.BlockSpec((R, EPBE * D), lambda s, n: (0, n)),
            pl.BlockSpec((1, EPBE, S_BLK), lambda s, n: (0, n, s)),
        ],
        out_specs=pl.BlockSpec((S_BLK, D), lambda s, n: (s, 0)),
        out_shape=jax.ShapeDtypeStruct((B * S, D), jnp.float32),
        compiler_params=pltpu.CompilerParams(
            dimension_semantics=("parallel", "arbitrary")),
    )(attn_out, eT, cwo_t)

    return out.reshape(B, S, D)


# speedup vs baseline: 2.1445x; 1.0227x over previous
import functools

import numpy as np
import jax
import jax.numpy as jnp
from jax import lax
from jax.experimental import pallas as pl
from jax.experimental.pallas import tpu as pltpu

S_BLK = 256


def _topk_cw(scores, nk):
    n = scores.shape[-1]
    iota = lax.broadcasted_iota(jnp.int32, scores.shape, 1)
    work = scores
    masks, tops = [], []
    for _ in range(nk):
        mk = jnp.max(work, axis=-1, keepdims=True)
        elig = work >= mk
        idx = jnp.min(jnp.where(elig, iota, n), axis=-1, keepdims=True)
        mask = iota == idx
        masks.append(mask)
        tops.append(mk)
        work = jnp.where(mask, -jnp.inf, work)
    m1 = tops[0]
    es = [jnp.exp(t - m1) for t in tops]
    denom = sum(es)
    cw = jnp.zeros_like(scores)
    for mask, e in zip(masks, es):
        cw = cw + jnp.where(mask, e / denom, 0.0)
    return cw


def _routing_body(x_ref, wt_ref, cw_ref, *, nk, n_routers, n_experts):
    scores = lax.dot_general(x_ref[...], wt_ref[...], (((1,), (1,)), ((), ())),
                             preferred_element_type=jnp.float32)
    for r in range(n_routers):
        s = scores[:, r * n_experts:(r + 1) * n_experts]
        cw_ref[r] = _topk_cw(s, nk)


def _cproj_body(x_ref, nb_ref, cw_ref, q_ref, k_ref, v_ref, *, n_experts, rp):
    proj = jnp.dot(x_ref[...], nb_ref[...], preferred_element_type=jnp.float32)
    rows = x_ref.shape[0]
    accs = [jnp.zeros((rows, rp), jnp.float32) for _ in range(3)]
    for e in range(n_experts):
        p = proj[:, e * rp:(e + 1) * rp]
        for r in range(3):
            accs[r] = accs[r] + cw_ref[r, :, e:e + 1] * p
    q_ref[...] = accs[0]
    k_ref[...] = accs[1]
    v_ref[...] = accs[2]


def _attn_body(q_ref, k_ref, v_ref, o_ref, *, n_heads, d_head, kc):
    qb = q_ref[...].astype(jnp.bfloat16)
    kb = k_ref[...].astype(jnp.bfloat16)
    v = v_ref[...]
    rows = q_ref.shape[0]
    nchunks = k_ref.shape[0] // kc
    scale = np.float32(1.0 / np.sqrt(d_head))
    outs = []
    for h in range(n_heads):
        qh = qb[:, h * d_head:(h + 1) * d_head]
        m = jnp.full((rows, 1), -jnp.inf, jnp.float32)
        l = jnp.zeros((rows, 1), jnp.float32)
        o = jnp.zeros((rows, d_head), jnp.float32)
        for j in range(nchunks):
            khc = kb[j * kc:(j + 1) * kc, h * d_head:(h + 1) * d_head]
            vhc = v[j * kc:(j + 1) * kc, h * d_head:(h + 1) * d_head]
            s = lax.dot_general(qh, khc, (((1,), (1,)), ((), ())),
                                preferred_element_type=jnp.float32) * scale
            mrow = jnp.max(s, axis=-1, keepdims=True)
            mnew = jnp.maximum(m, mrow)
            corr = jnp.where(m == mnew, np.float32(0.0), m - mnew)
            p = jnp.exp(s - mnew)
            ec = jnp.exp(corr)
            lnew = ec * l + jnp.sum(p, axis=-1, keepdims=True)
            oacc = (ec * l) * o
            omm = oacc + jnp.dot(p, vhc, preferred_element_type=jnp.float32)
            o = omm * (np.float32(1.0) / lnew)
            m, l = mnew, lnew
        outs.append(o)
    o_ref[...] = jnp.concatenate(outs, axis=1)


def _eproj_body(a_ref, nb_ref, cw_ref, o_ref, *, epb, d_model):
    proj = jnp.dot(a_ref[...], nb_ref[...], preferred_element_type=jnp.float32)

    @pl.when(pl.program_id(1) == 0)
    def _():
        o_ref[...] = jnp.zeros_like(o_ref)

    acc = o_ref[...]
    for i in range(epb):
        p = proj[:, i * d_model:(i + 1) * d_model]
        acc = acc + cw_ref[0, :, i:i + 1] * p
    o_ref[...] = acc


def kernel(x, compress_neurons, expand_neurons, Wq, Wk, Wv, Wo):
    B, S, D = x.shape
    N, _, R = compress_neurons.shape
    NE = expand_neurons.shape[0]
    n_heads = 12
    d_head = R // n_heads
    xs = x.reshape(B * S, D)
    n_sb = (B * S) // S_BLK

    wt = jnp.concatenate([Wq, Wk, Wv], axis=0)
    cw = pl.pallas_call(
        functools.partial(_routing_body, nk=4, n_routers=3, n_experts=N),
        grid=(n_sb,),
        in_specs=[
            pl.BlockSpec((S_BLK, D), lambda s: (s, 0)),
            pl.BlockSpec((3 * N, D), lambda s: (0, 0)),
        ],
        out_specs=pl.BlockSpec((3, S_BLK, N), lambda s: (0, s, 0)),
        out_shape=jax.ShapeDtypeStruct((3, B * S, N), jnp.float32),
    )(xs, wt)

    RP = 128
    nT = jnp.pad(compress_neurons.transpose(1, 0, 2), ((0, 0), (0, 0), (0, RP - R))
                 ).reshape(D, N * RP)
    qkv_spec = pl.BlockSpec((S_BLK, RP), lambda s: (s, 0))
    qp, kp, vp = pl.pallas_call(
        functools.partial(_cproj_body, n_experts=N, rp=RP),
        grid=(n_sb,),
        in_specs=[
            pl.BlockSpec((S_BLK, D), lambda s: (s, 0)),
            pl.BlockSpec((D, N * RP), lambda s: (0, 0)),
            pl.BlockSpec((3, S_BLK, N), lambda s: (0, s, 0)),
        ],
        out_specs=[qkv_spec, qkv_spec, qkv_spec],
        out_shape=[jax.ShapeDtypeStruct((B * S, RP), jnp.float32)] * 3,
    )(xs, nT, cw)
    q, k, v = qp[:, :R], kp[:, :R], vp[:, :R]

    attn_out = pl.pallas_call(
        functools.partial(_attn_body, n_heads=n_heads, d_head=d_head, kc=1024),
        grid=(n_sb,),
        in_specs=[
            pl.BlockSpec((S_BLK, R), lambda s: (s, 0)),
            pl.BlockSpec((B * S, R), lambda s: (0, 0)),
            pl.BlockSpec((B * S, R), lambda s: (0, 0)),
        ],
        out_specs=pl.BlockSpec((S_BLK, R), lambda s: (s, 0)),
        out_shape=jax.ShapeDtypeStruct((B * S, R), jnp.float32),
    )(q, k, v)

    cwo = pl.pallas_call(
        functools.partial(_routing_body, nk=2, n_routers=1, n_experts=NE),
        grid=(n_sb,),
        in_specs=[
            pl.BlockSpec((S_BLK, R), lambda s: (s, 0)),
            pl.BlockSpec((NE, R), lambda s: (0, 0)),
        ],
        out_specs=pl.BlockSpec((1, S_BLK, NE), lambda s: (0, s, 0)),
        out_shape=jax.ShapeDtypeStruct((1, B * S, NE), jnp.float32),
    )(attn_out, Wo)

    EPBE = 16
    n_nb = NE // EPBE
    cwr = cwo[0].reshape(B * S, n_nb, EPBE).transpose(1, 0, 2)
    eT = expand_neurons.transpose(1, 0, 2).reshape(R, NE * D)
    out = pl.pallas_call(
        functools.partial(_eproj_body, epb=EPBE, d_model=D),
        grid=(n_sb, n_nb),
        in_specs=[
            pl.BlockSpec((S_BLK, R), lambda s, n: (s, 0)),
            pl.BlockSpec((R, EPBE * D), lambda s, n: (0, n)),
            pl.BlockSpec((1, S_BLK, EPBE), lambda s, n: (n, s, 0)),
        ],
        out_specs=pl.BlockSpec((S_BLK, D), lambda s, n: (s, 0)),
        out_shape=jax.ShapeDtypeStruct((B * S, D), jnp.float32),
        compiler_params=pltpu.CompilerParams(
            dimension_semantics=("parallel", "arbitrary")),
    )(attn_out, eT, cwr)

    return out.reshape(B, S, D)
